# Initial kernel scaffold; baseline (speedup 1.0000x reference)
import jax
import jax.numpy as jnp
from jax.experimental import pallas as pl


def kernel(x, edge_index, emb_W, emb_b, pre_W1, pre_b1, post_W1, post_b1, lin_W1, lin_b1, bn_gamma, bn_beta, pre_W2, pre_b2, post_W2, post_b2, lin_W2, lin_b2):
    raise NotImplementedError("write your pallas kernel here")


# speedup vs baseline: 1.1119x; 1.1119x over previous
import functools
import numpy as np
import jax
import jax.numpy as jnp
from jax.experimental import pallas as pl
from jax.experimental.pallas import tpu as pltpu

AVG_LOG_CONST = float(np.log(33.0))


def _mm_kernel(x_ref, w_ref, o_ref):
    o_ref[...] = jnp.dot(x_ref[...], w_ref[...], preferred_element_type=jnp.float32)


def _mm(x, w, bn=512):
    n, k = x.shape
    m = w.shape[1]
    pad = (-n) % bn
    if pad:
        x = jnp.pad(x, ((0, pad), (0, 0)))
    grid = ((n + pad) // bn,)
    out = pl.pallas_call(
        _mm_kernel,
        grid=grid,
        in_specs=[
            pl.BlockSpec((bn, k), lambda i: (i, 0)),
            pl.BlockSpec((k, m), lambda i: (0, 0)),
        ],
        out_specs=pl.BlockSpec((bn, m), lambda i: (i, 0)),
        out_shape=jax.ShapeDtypeStruct((n + pad, m), jnp.float32),
    )(x, w)
    return out[:n]


def _pna_conv(x, src, dst, cnt, pre_W, pre_b, post_W, post_b, lin_W, lin_b):
    n, F = x.shape
    towers = pre_W.shape[0]
    deg_c = jnp.maximum(cnt, 1.0)[:, None]
    scale_amp = jnp.log(deg_c + 1.0) / AVG_LOG_CONST
    scale_att = AVG_LOG_CONST / jnp.log(deg_c + 1.0)
    has = (cnt > 0)[:, None]
    tower_outs = []
    for t in range(towers):
        A = _mm(x, pre_W[t][:F]) + pre_b[t]
        B = _mm(x, pre_W[t][F:])
        Bs = B[src]
        SB = jax.ops.segment_sum(Bs, dst, num_segments=n)
        SB2 = jax.ops.segment_sum(Bs * Bs, dst, num_segments=n)
        MN = jax.ops.segment_min(Bs, dst, num_segments=n)
        MX = jax.ops.segment_max(Bs, dst, num_segments=n)
        mean = (cnt[:, None] * A + SB) / deg_c
        var = SB2 / deg_c - (SB / deg_c) ** 2
        std = jnp.sqrt(jax.nn.relu(var) + 1e-5)
        mn = jnp.where(has, A + MN, 0.0)
        mx = jnp.where(has, A + MX, 0.0)
        agg = jnp.concatenate([mean, mn, mx, std], axis=-1)
        W0 = post_W[t][:F]
        Wc = jnp.concatenate([post_W[t][F:F + 4 * F], post_W[t][F + 4 * F:F + 8 * F], post_W[t][F + 8 * F:]], axis=1)
        U = _mm(agg, Wc)
        fo = post_W.shape[2]
        o = _mm(x, W0) + U[:, :fo] + scale_amp * U[:, fo:2 * fo] + scale_att * U[:, 2 * fo:] + post_b[t]
        tower_outs.append(o)
    out = jnp.concatenate(tower_outs, axis=-1)
    return _mm(out, lin_W) + lin_b


def kernel(x, edge_index, emb_W, emb_b, pre_W1, pre_b1, post_W1, post_b1, lin_W1, lin_b1, bn_gamma, bn_beta, pre_W2, pre_b2, post_W2, post_b2, lin_W2, lin_b2):
    src, dst = edge_index[0], edge_index[1]
    n = x.shape[0]
    ones = jnp.ones((src.shape[0],), x.dtype)
    cnt = jax.ops.segment_sum(ones, dst, num_segments=n)
    h = _mm(x, emb_W) + emb_b
    h = _pna_conv(h, src, dst, cnt, pre_W1, pre_b1, post_W1, post_b1, lin_W1, lin_b1)
    h = h / jnp.sqrt(1.0 + 1e-5) * bn_gamma + bn_beta
    h = jax.nn.relu(h)
    h = _pna_conv(h, src, dst, cnt, pre_W2, pre_b2, post_W2, post_b2, lin_W2, lin_b2)
    return h


# speedup vs baseline: 5.1302x; 4.6138x over previous
import functools
import numpy as np
import jax
import jax.numpy as jnp
from jax import lax
from jax.experimental import pallas as pl
from jax.experimental.pallas import tpu as pltpu
from jax.experimental.pallas import tpu_sc as plsc

AVG_LOG_CONST = float(np.log(33.0))

NW = 32
NPB = 320
NPAD = NW * NPB
ROWS = NPB + 1
TRASH = NPB
K = 128
FLUSH = 8032
BUF = 8192
SENT_VREGS = 9
CH = 2000

_MESH = plsc.VectorSubcoreMesh(core_axis_name="c", subcore_axis_name="s")


def _k0_body(src_hbm, dst_hbm, list_hbm, cnt_hbm, svm, dvm, buf, cvm):
    w = lax.axis_index("s") * 2 + lax.axis_index("c")
    lo = w * NPB
    hi = lo + NPB
    e_total = src_hbm.shape[0]
    n_ch = e_total // CH
    lanes = lax.iota(jnp.int32, 16)

    def vreg_body(i, carry):
        fill, woff = carry
        s = svm[pl.ds(i * 16, 16)]
        d = dvm[pl.ds(i * 16, 16)]
        m = (d >= lo) & (d < hi)
        packed = (s << 9) | (d - lo)
        _, sv = plsc.sort_key_val(jnp.where(m, 0, 1), packed)
        buf[pl.ds(fill, 16)] = sv
        fill = fill + plsc.all_reduce_population_count(m)[0]

        def do_flush(args):
            f, wo = args
            wo8 = pl.multiple_of(wo, 8)
            pltpu.sync_copy(buf.at[pl.ds(0, FLUSH)], list_hbm.at[w, pl.ds(wo8, FLUSH)])
            buf[pl.ds(0, 16)] = buf[pl.ds(FLUSH, 16)]
            return f - FLUSH, wo + FLUSH

        return lax.cond(fill >= FLUSH, do_flush, lambda a: a, (fill, woff))

    def chunk_body(c, carry):
        pltpu.sync_copy(src_hbm.at[pl.ds(c * CH, CH)], svm)
        pltpu.sync_copy(dst_hbm.at[pl.ds(c * CH, CH)], dvm)
        return lax.fori_loop(0, CH // 16, vreg_body, carry)

    fill, woff = lax.fori_loop(0, n_ch, chunk_body, (jnp.int32(0), jnp.int32(0)))

    sent = jnp.full((16,), TRASH, jnp.int32)
    for i in range(SENT_VREGS):
        buf[pl.ds(fill + i * 16, 16)] = sent
    pltpu.sync_copy(buf.at[pl.ds(0, BUF)], list_hbm.at[w, pl.ds(pl.multiple_of(woff, 8), BUF)])
    cvm[...] = jnp.full((16,), woff + fill, jnp.int32)
    pltpu.sync_copy(cvm, cnt_hbm.at[w])


def _bin_edges(src, dst):
    e_total = src.shape[0]
    cap = e_total + BUF
    k0 = functools.partial(
        pl.kernel,
        out_type=[
            jax.ShapeDtypeStruct((NW, cap), jnp.int32),
            jax.ShapeDtypeStruct((NW, 16), jnp.int32),
        ],
        mesh=_MESH,
        compiler_params=pltpu.CompilerParams(use_tc_tiling_on_sc=False, needs_layout_passes=False),
        scratch_types=[
            pltpu.VMEM((CH,), jnp.int32),
            pltpu.VMEM((CH,), jnp.int32),
            pltpu.VMEM((BUF + 16,), jnp.int32),
            pltpu.VMEM((16,), jnp.int32),
        ],
    )(_k0_body)
    return k0(src, dst)


def _k1_body(list_hbm, cnts_hbm, bcat_hbm,
             sb_o, sq_o, mn_o, mx_o, cnt_o,
             wbufs, ibufs, dbufs, rowss, acc, cacc, cvm, gsems, lsems):
    w = lax.axis_index("s") * 2 + lax.axis_index("c")
    lo = w * NPB

    pltpu.sync_copy(cnts_hbm.at[w], cvm)
    count = cvm[pl.ds(0, 16)][0]
    nchunks = lax.div(count + (K - 1), K)
    npairs = lax.div(nchunks + 1, 2)

    zero = jnp.zeros((16,), jnp.float32)
    big = jnp.full((16,), 3e38, jnp.float32)
    onehot = jnp.where(lax.iota(jnp.int32, 16) == 0, 1.0, 0.0).astype(jnp.float32)

    def initc(j, t):
        cacc[j, :] = zero
        return t

    lax.fori_loop(0, ROWS, initc, 0)

    def unpack(wb, ib, db, p):
        base = p * NPAD
        for i in range(K // 16):
            sl = pl.ds(i * 16, 16)
            wv = wb[sl]
            ib[sl] = lax.shift_right_logical(wv, 9) + base
            db[sl] = wv & 511

    def process(rw, db):
        def edges16(it, t):
            e0 = it * 16
            dlv = db[pl.ds(e0, 16)]
            for k in range(16):
                e = e0 + k
                dl = dlv[k]
                for g in range(4):
                    sl = pl.ds(g * 16, 16)
                    r = rw[e, sl]
                    plsc.addupdate(acc.at[0, dl, sl], r)
                    plsc.addupdate(acc.at[1, dl, sl], r * r)
                    acc[2, dl, sl] = jnp.minimum(acc[2, dl, sl], r)
                    acc[3, dl, sl] = jnp.maximum(acc[3, dl, sl], r)
                plsc.addupdate(cacc.at[dl], onehot)
            return t

        lax.fori_loop(0, K // 16, edges16, 0)

    def pass_body(p, carry0):
        def initrow(j, t):
            for g in range(4):
                sl = pl.ds(g * 16, 16)
                acc[0, j, sl] = zero
                acc[1, j, sl] = zero
                acc[2, j, sl] = big
                acc[3, j, sl] = -big
            return t

        lax.fori_loop(0, ROWS, initrow, 0)

        @pl.when(nchunks > 0)
        def _():
            pltpu.sync_copy(list_hbm.at[w, pl.ds(0, K)], wbufs[0])
            unpack(wbufs[0], ibufs[0], dbufs[0], p)
            pltpu.make_async_copy(bcat_hbm.at[ibufs[0]], rowss[0], gsems[0]).start()

        @pl.when(nchunks > 1)
        def _():
            pltpu.make_async_copy(list_hbm.at[w, pl.ds(K, K)], wbufs[1], lsems[1]).start()

        def pair_body(gidx, t):
            for par in range(2):
                c = gidx * 2 + par
                npar = 1 - par

                @pl.when(c < nchunks)
                def _():
                    @pl.when(c + 1 < nchunks)
                    def _():
                        pltpu.make_async_copy(
                            list_hbm.at[w, pl.ds(pl.multiple_of((c + 1) * K, 8), K)],
                            wbufs[npar], lsems[npar]
                        ).wait()
                        unpack(wbufs[npar], ibufs[npar], dbufs[npar], p)
                        pltpu.make_async_copy(
                            bcat_hbm.at[ibufs[npar]], rowss[npar], gsems[npar]
                        ).start()

                    @pl.when(c + 2 < nchunks)
                    def _():
                        pltpu.make_async_copy(
                            list_hbm.at[w, pl.ds(pl.multiple_of((c + 2) * K, 8), K)],
                            wbufs[par], lsems[par]
                        ).start()

                    pltpu.make_async_copy(bcat_hbm.at[ibufs[par]], rowss[par], gsems[par]).wait()
                    process(rowss[par], dbufs[par])
            return t

        lax.fori_loop(0, npairs, pair_body, 0)

        for s, out in enumerate((sb_o, sq_o, mn_o, mx_o)):
            pltpu.sync_copy(
                acc.at[s, pl.ds(0, NPB), :],
                out.at[pl.ds(lo, NPB), pl.ds(p * 64, 64)],
            )
        return carry0

    lax.fori_loop(0, 4, pass_body, 0)
    pltpu.sync_copy(cacc.at[pl.ds(0, NPB), :], cnt_o.at[pl.ds(lo, NPB), :])


def _scatter_stats(lists, cnts, bcat):
    k1 = functools.partial(
        pl.kernel,
        out_type=[
            jax.ShapeDtypeStruct((NPAD, 256), jnp.float32),
            jax.ShapeDtypeStruct((NPAD, 256), jnp.float32),
            jax.ShapeDtypeStruct((NPAD, 256), jnp.float32),
            jax.ShapeDtypeStruct((NPAD, 256), jnp.float32),
            jax.ShapeDtypeStruct((NPAD, 16), jnp.float32),
        ],
        mesh=_MESH,
        compiler_params=pltpu.CompilerParams(use_tc_tiling_on_sc=False, needs_layout_passes=False),
        scratch_types=[
            [pltpu.VMEM((K,), jnp.int32)] * 2,
            [pltpu.VMEM((K,), jnp.int32)] * 2,
            [pltpu.VMEM((K,), jnp.int32)] * 2,
            [pltpu.VMEM((K, 64), jnp.float32)] * 2,
            pltpu.VMEM((4, ROWS, 64), jnp.float32),
            pltpu.VMEM((ROWS, 16), jnp.float32),
            pltpu.VMEM((16,), jnp.int32),
            [pltpu.SemaphoreType.DMA] * 2,
            [pltpu.SemaphoreType.DMA] * 2,
        ],
    )(_k1_body)
    return k1(lists, cnts, bcat)


def _mm_kernel(x_ref, w_ref, o_ref):
    o_ref[...] = jnp.dot(x_ref[...], w_ref[...], preferred_element_type=jnp.float32)


def _mm_bias_kernel(x_ref, w_ref, b_ref, o_ref):
    o_ref[...] = (
        jnp.dot(x_ref[...], w_ref[...], preferred_element_type=jnp.float32) + b_ref[...]
    )


def _mm(x, w, b=None, bn=512):
    n, kdim = x.shape
    m = w.shape[1]
    grid = (n // bn,)
    if b is None:
        return pl.pallas_call(
            _mm_kernel,
            grid=grid,
            in_specs=[
                pl.BlockSpec((bn, kdim), lambda i: (i, 0)),
                pl.BlockSpec((kdim, m), lambda i: (0, 0)),
            ],
            out_specs=pl.BlockSpec((bn, m), lambda i: (i, 0)),
            out_shape=jax.ShapeDtypeStruct((n, m), jnp.float32),
        )(x, w)
    return pl.pallas_call(
        _mm_bias_kernel,
        grid=grid,
        in_specs=[
            pl.BlockSpec((bn, kdim), lambda i: (i, 0)),
            pl.BlockSpec((kdim, m), lambda i: (0, 0)),
            pl.BlockSpec((1, m), lambda i: (0, 0)),
        ],
        out_specs=pl.BlockSpec((bn, m), lambda i: (i, 0)),
        out_shape=jax.ShapeDtypeStruct((n, m), jnp.float32),
    )(x, w, b.reshape(1, m))


def _post_body(fo, with_bn, h_ref, a_ref, sb_ref, sq_ref, mn_ref, mx_ref, cnt_ref,
               w0_ref, wc_ref, preb_ref, postb_ref, linw_ref, linb_ref, gamma_ref,
               beta_ref, o_ref):
    cnt = cnt_ref[...] * 0.25
    deg = jnp.maximum(cnt, 1.0)
    lg = jnp.log(deg + 1.0)
    amp = lg * (1.0 / AVG_LOG_CONST)
    att = AVG_LOG_CONST / lg
    has = cnt > 0.0
    h = h_ref[...]
    ob = jnp.dot(h, w0_ref[...], preferred_element_type=jnp.float32) + postb_ref[...]
    outs = []
    for t in range(2):
        cs = pl.ds(t * 128, 128)
        A = a_ref[:, cs] + preb_ref[0, cs]
        SB = sb_ref[:, cs]
        SQ = sq_ref[:, cs]
        MN = mn_ref[:, cs]
        MX = mx_ref[:, cs]
        mean = (cnt * A + SB) / deg
        q = SB / deg
        var = SQ / deg - q * q
        std = jnp.sqrt(jnp.maximum(var, 0.0) + 1e-5)
        mn = jnp.where(has, A + MN, 0.0)
        mx = jnp.where(has, A + MX, 0.0)
        agg = jnp.concatenate([mean, mn, mx, std], axis=-1)
        U = jnp.dot(agg, wc_ref[t], preferred_element_type=jnp.float32)
        o_t = (
            ob[:, t * fo:(t + 1) * fo]
            + U[:, 0:fo]
            + amp * U[:, fo:2 * fo]
            + att * U[:, 2 * fo:3 * fo]
        )
        outs.append(o_t)
    o = jnp.concatenate(outs, axis=-1)
    o = jnp.dot(o, linw_ref[...], preferred_element_type=jnp.float32) + linb_ref[...]
    if with_bn:
        o = o * (1.0 / np.sqrt(1.0 + 1e-5)) * gamma_ref[...] + beta_ref[...]
        o = jnp.maximum(o, 0.0)
    o_ref[...] = o


def _post(h, A, SB, SQ, MN, MX, cnt, w0, wc, preb, postb, linw, linb, gamma, beta,
          with_bn, bn=512):
    n = h.shape[0]
    fo = w0.shape[1] // 2
    fout = linw.shape[1]
    grid = (n // bn,)
    row = lambda shp: pl.BlockSpec(shp, lambda i: (i, 0))
    full2 = lambda shp: pl.BlockSpec(shp, lambda i: (0, 0))
    return pl.pallas_call(
        functools.partial(_post_body, fo, with_bn),
        grid=grid,
        in_specs=[
            row((bn, 128)),
            row((bn, 256)),
            row((bn, 256)),
            row((bn, 256)),
            row((bn, 256)),
            row((bn, 256)),
            row((bn, 1)),
            full2((128, 2 * fo)),
            pl.BlockSpec((2, 512, 3 * fo), lambda i: (0, 0, 0)),
            full2((1, 256)),
            full2((1, 2 * fo)),
            full2((fout, fout)),
            full2((1, fout)),
            full2((1, fout)),
            full2((1, fout)),
        ],
        out_specs=row((bn, fout)),
        out_shape=jax.ShapeDtypeStruct((n, fout), jnp.float32),
    )(h, A, SB, SQ, MN, MX, cnt, w0, wc, preb, postb, linw, linb, gamma, beta)


def _conv(h, lists, cnts, pre_W, pre_b, post_W, post_b, lin_W, lin_b,
          gamma, beta, with_bn):
    F = 128
    fo = post_W.shape[2]
    Wd = jnp.concatenate([pre_W[0][:F], pre_W[1][:F]], axis=1)
    Ws = jnp.concatenate([pre_W[0][F:], pre_W[1][F:]], axis=1)
    A = _mm(h, Wd)
    B = _mm(h, Ws)
    bcat = B.reshape(NPAD, 4, 64).transpose(1, 0, 2).reshape(4 * NPAD, 64)
    SB, SQ, MN, MX, cnt16 = _scatter_stats(lists, cnts, bcat)
    cnt = cnt16[:, 0:1]
    w0 = jnp.concatenate([post_W[0][:F], post_W[1][:F]], axis=1)
    wc = jnp.stack([
        jnp.concatenate([post_W[t][F:F + 512], post_W[t][F + 512:F + 1024],
                         post_W[t][F + 1024:]], axis=1)
        for t in range(2)
    ])
    preb = jnp.concatenate([pre_b[0], pre_b[1]])[None, :]
    postb = jnp.concatenate([post_b[0], post_b[1]])[None, :]
    fout = lin_W.shape[1]
    if gamma is None:
        gamma = jnp.ones((fout,), jnp.float32)
        beta = jnp.zeros((fout,), jnp.float32)
    return _post(h, A, SB, SQ, MN, MX, cnt, w0, wc, preb, postb, lin_W,
                 lin_b.reshape(1, -1), gamma.reshape(1, -1), beta.reshape(1, -1), with_bn)


def kernel(x, edge_index, emb_W, emb_b, pre_W1, pre_b1, post_W1, post_b1, lin_W1,
           lin_b1, bn_gamma, bn_beta, pre_W2, pre_b2, post_W2, post_b2, lin_W2, lin_b2):
    n = x.shape[0]
    src = edge_index[0]
    dst = edge_index[1]
    lists, cnts = _bin_edges(src, dst)
    x_pad = jnp.pad(x, ((0, NPAD - n), (0, 0)))
    h = _mm(x_pad, emb_W, emb_b)
    h = _conv(h, lists, cnts, pre_W1, pre_b1, post_W1, post_b1, lin_W1, lin_b1,
              bn_gamma, bn_beta, True)
    h = _conv(h, lists, cnts, pre_W2, pre_b2, post_W2, post_b2, lin_W2, lin_b2,
              None, None, False)
    return h[:n]


# speedup vs baseline: 5.2112x; 1.0158x over previous
import functools
import numpy as np
import jax
import jax.numpy as jnp
from jax import lax
from jax.experimental import pallas as pl
from jax.experimental.pallas import tpu as pltpu
from jax.experimental.pallas import tpu_sc as plsc

AVG_LOG_CONST = float(np.log(33.0))

NW = 32
NPB = 320
NPAD = NW * NPB
ROWS = NPB + 1
TRASH = NPB
K = 256
FLUSH = 6144
BUF = 8192
SENT_VREGS = 18
CH = 2000

_MESH = plsc.VectorSubcoreMesh(core_axis_name="c", subcore_axis_name="s")


def _k0_body(src_hbm, dst_hbm, list_hbm, cnt_hbm, svm, dvm, buf, cvm):
    w = lax.axis_index("s") * 2 + lax.axis_index("c")
    lo = w * NPB
    hi = lo + NPB
    e_total = src_hbm.shape[0]
    n_ch = e_total // CH
    lanes = lax.iota(jnp.int32, 16)

    def vreg_body(i, fill):
        s = svm[pl.ds(i * 16, 16)]
        d = dvm[pl.ds(i * 16, 16)]
        m = (d >= lo) & (d < hi)
        packed = (s << 9) | (d - lo)
        _, sv = plsc.sort_key_val(jnp.where(m, 0, 1), packed)
        buf[pl.ds(fill, 16)] = sv
        return fill + plsc.all_reduce_population_count(m)[0]

    def chunk_body(c, carry):
        fill, woff = carry
        pltpu.sync_copy(src_hbm.at[pl.ds(c * CH, CH)], svm)
        pltpu.sync_copy(dst_hbm.at[pl.ds(c * CH, CH)], dvm)
        fill = lax.fori_loop(0, CH // 16, vreg_body, fill)

        def do_flush(args):
            f, wo = args
            wo8 = pl.multiple_of(wo, 8)
            pltpu.sync_copy(buf.at[pl.ds(0, FLUSH)], list_hbm.at[w, pl.ds(wo8, FLUSH)])
            rem = f - FLUSH

            def mv(j, t):
                buf[pl.ds(j * 16, 16)] = buf[pl.ds(FLUSH + j * 16, 16)]
                return t

            lax.fori_loop(0, lax.div(rem + 15, 16), mv, 0)
            return rem, wo + FLUSH

        return lax.cond(fill >= FLUSH, do_flush, lambda a: a, (fill, woff))

    fill, woff = lax.fori_loop(0, n_ch, chunk_body, (jnp.int32(0), jnp.int32(0)))

    sent = jnp.full((16,), TRASH, jnp.int32)
    for i in range(SENT_VREGS):
        buf[pl.ds(fill + i * 16, 16)] = sent
    pltpu.sync_copy(buf.at[pl.ds(0, BUF)], list_hbm.at[w, pl.ds(pl.multiple_of(woff, 8), BUF)])
    cvm[...] = jnp.full((16,), woff + fill, jnp.int32)
    pltpu.sync_copy(cvm, cnt_hbm.at[w])


def _bin_edges(src, dst):
    e_total = src.shape[0]
    cap = e_total + BUF
    k0 = functools.partial(
        pl.kernel,
        out_type=[
            jax.ShapeDtypeStruct((NW, cap), jnp.int32),
            jax.ShapeDtypeStruct((NW, 16), jnp.int32),
        ],
        mesh=_MESH,
        compiler_params=pltpu.CompilerParams(use_tc_tiling_on_sc=False, needs_layout_passes=False),
        scratch_types=[
            pltpu.VMEM((CH,), jnp.int32),
            pltpu.VMEM((CH,), jnp.int32),
            pltpu.VMEM((BUF + 16,), jnp.int32),
            pltpu.VMEM((16,), jnp.int32),
        ],
    )(_k0_body)
    return k0(src, dst)


def _k1_body(list_hbm, cnts_hbm, bcat_hbm,
             sb_o, sq_o, mn_o, mx_o, cnt_o,
             wbufs, ibufs, dbufs, rowss, acc, cacc, cvm, gsems, lsems):
    w = lax.axis_index("s") * 2 + lax.axis_index("c")
    lo = w * NPB

    pltpu.sync_copy(cnts_hbm.at[w], cvm)
    count = cvm[pl.ds(0, 16)][0]
    nchunks = lax.div(count + (K - 1), K)
    npairs = lax.div(nchunks + 1, 2)

    zero = jnp.zeros((16,), jnp.float32)
    big = jnp.full((16,), 3e38, jnp.float32)
    onehot = jnp.where(lax.iota(jnp.int32, 16) == 0, 1.0, 0.0).astype(jnp.float32)

    def initc(j, t):
        cacc[j, :] = zero
        return t

    lax.fori_loop(0, ROWS, initc, 0)

    def unpack(wb, ib, db, p):
        base = p * NPAD
        for i in range(K // 16):
            sl = pl.ds(i * 16, 16)
            wv = wb[sl]
            ib[sl] = lax.shift_right_logical(wv, 9) + base
            db[sl] = wv & 511

    def process(rw, db):
        def edges16(it, t):
            e0 = it * 16
            dlv = db[pl.ds(e0, 16)]
            for k in range(16):
                e = e0 + k
                dl = dlv[k]
                for g in range(4):
                    sl = pl.ds(g * 16, 16)
                    r = rw[e, sl]
                    plsc.addupdate(acc.at[0, dl, sl], r)
                    plsc.addupdate(acc.at[1, dl, sl], r * r)
                    acc[2, dl, sl] = jnp.minimum(acc[2, dl, sl], r)
                    acc[3, dl, sl] = jnp.maximum(acc[3, dl, sl], r)
                plsc.addupdate(cacc.at[dl], onehot)
            return t

        lax.fori_loop(0, K // 16, edges16, 0)

    def pass_body(p, carry0):
        def initrow(j, t):
            for g in range(4):
                sl = pl.ds(g * 16, 16)
                acc[0, j, sl] = zero
                acc[1, j, sl] = zero
                acc[2, j, sl] = big
                acc[3, j, sl] = -big
            return t

        lax.fori_loop(0, ROWS, initrow, 0)

        @pl.when(nchunks > 0)
        def _():
            pltpu.sync_copy(list_hbm.at[w, pl.ds(0, K)], wbufs[0])
            unpack(wbufs[0], ibufs[0], dbufs[0], p)
            pltpu.make_async_copy(bcat_hbm.at[ibufs[0]], rowss[0], gsems[0]).start()

        @pl.when(nchunks > 1)
        def _():
            pltpu.make_async_copy(list_hbm.at[w, pl.ds(K, K)], wbufs[1], lsems[1]).start()

        def pair_body(gidx, t):
            for par in range(2):
                c = gidx * 2 + par
                npar = 1 - par

                @pl.when(c < nchunks)
                def _():
                    @pl.when(c + 1 < nchunks)
                    def _():
                        pltpu.make_async_copy(
                            list_hbm.at[w, pl.ds(pl.multiple_of((c + 1) * K, 8), K)],
                            wbufs[npar], lsems[npar]
                        ).wait()
                        unpack(wbufs[npar], ibufs[npar], dbufs[npar], p)
                        pltpu.make_async_copy(
                            bcat_hbm.at[ibufs[npar]], rowss[npar], gsems[npar]
                        ).start()

                    @pl.when(c + 2 < nchunks)
                    def _():
                        pltpu.make_async_copy(
                            list_hbm.at[w, pl.ds(pl.multiple_of((c + 2) * K, 8), K)],
                            wbufs[par], lsems[par]
                        ).start()

                    pltpu.make_async_copy(bcat_hbm.at[ibufs[par]], rowss[par], gsems[par]).wait()
                    process(rowss[par], dbufs[par])
            return t

        lax.fori_loop(0, npairs, pair_body, 0)

        for s, out in enumerate((sb_o, sq_o, mn_o, mx_o)):
            pltpu.sync_copy(
                acc.at[s, pl.ds(0, NPB), :],
                out.at[pl.ds(lo, NPB), pl.ds(p * 64, 64)],
            )
        return carry0

    lax.fori_loop(0, 4, pass_body, 0)
    pltpu.sync_copy(cacc.at[pl.ds(0, NPB), :], cnt_o.at[pl.ds(lo, NPB), :])


def _scatter_stats(lists, cnts, bcat):
    k1 = functools.partial(
        pl.kernel,
        out_type=[
            jax.ShapeDtypeStruct((NPAD, 256), jnp.float32),
            jax.ShapeDtypeStruct((NPAD, 256), jnp.float32),
            jax.ShapeDtypeStruct((NPAD, 256), jnp.float32),
            jax.ShapeDtypeStruct((NPAD, 256), jnp.float32),
            jax.ShapeDtypeStruct((NPAD, 16), jnp.float32),
        ],
        mesh=_MESH,
        compiler_params=pltpu.CompilerParams(use_tc_tiling_on_sc=False, needs_layout_passes=False),
        scratch_types=[
            [pltpu.VMEM((K,), jnp.int32)] * 2,
            [pltpu.VMEM((K,), jnp.int32)] * 2,
            [pltpu.VMEM((K,), jnp.int32)] * 2,
            [pltpu.VMEM((K, 64), jnp.float32)] * 2,
            pltpu.VMEM((4, ROWS, 64), jnp.float32),
            pltpu.VMEM((ROWS, 16), jnp.float32),
            pltpu.VMEM((16,), jnp.int32),
            [pltpu.SemaphoreType.DMA] * 2,
            [pltpu.SemaphoreType.DMA] * 2,
        ],
    )(_k1_body)
    return k1(lists, cnts, bcat)


def _mm_kernel(x_ref, w_ref, o_ref):
    o_ref[...] = jnp.dot(x_ref[...], w_ref[...], preferred_element_type=jnp.float32)


def _mm_bias_kernel(x_ref, w_ref, b_ref, o_ref):
    o_ref[...] = (
        jnp.dot(x_ref[...], w_ref[...], preferred_element_type=jnp.float32) + b_ref[...]
    )


def _mm(x, w, b=None, bn=512):
    n, kdim = x.shape
    m = w.shape[1]
    grid = (n // bn,)
    if b is None:
        return pl.pallas_call(
            _mm_kernel,
            grid=grid,
            in_specs=[
                pl.BlockSpec((bn, kdim), lambda i: (i, 0)),
                pl.BlockSpec((kdim, m), lambda i: (0, 0)),
            ],
            out_specs=pl.BlockSpec((bn, m), lambda i: (i, 0)),
            out_shape=jax.ShapeDtypeStruct((n, m), jnp.float32),
        )(x, w)
    return pl.pallas_call(
        _mm_bias_kernel,
        grid=grid,
        in_specs=[
            pl.BlockSpec((bn, kdim), lambda i: (i, 0)),
            pl.BlockSpec((kdim, m), lambda i: (0, 0)),
            pl.BlockSpec((1, m), lambda i: (0, 0)),
        ],
        out_specs=pl.BlockSpec((bn, m), lambda i: (i, 0)),
        out_shape=jax.ShapeDtypeStruct((n, m), jnp.float32),
    )(x, w, b.reshape(1, m))


def _post_body(fo, with_bn, h_ref, a_ref, sb_ref, sq_ref, mn_ref, mx_ref, cnt_ref,
               w0_ref, wc_ref, preb_ref, postb_ref, linw_ref, linb_ref, gamma_ref,
               beta_ref, o_ref):
    cnt = cnt_ref[...] * 0.25
    deg = jnp.maximum(cnt, 1.0)
    lg = jnp.log(deg + 1.0)
    amp = lg * (1.0 / AVG_LOG_CONST)
    att = AVG_LOG_CONST / lg
    has = cnt > 0.0
    h = h_ref[...]
    ob = jnp.dot(h, w0_ref[...], preferred_element_type=jnp.float32) + postb_ref[...]
    outs = []
    for t in range(2):
        cs = pl.ds(t * 128, 128)
        A = a_ref[:, cs] + preb_ref[0, cs]
        SB = sb_ref[:, cs]
        SQ = sq_ref[:, cs]
        MN = mn_ref[:, cs]
        MX = mx_ref[:, cs]
        mean = (cnt * A + SB) / deg
        q = SB / deg
        var = SQ / deg - q * q
        std = jnp.sqrt(jnp.maximum(var, 0.0) + 1e-5)
        mn = jnp.where(has, A + MN, 0.0)
        mx = jnp.where(has, A + MX, 0.0)
        agg = jnp.concatenate([mean, mn, mx, std], axis=-1)
        U = jnp.dot(agg, wc_ref[t], preferred_element_type=jnp.float32)
        o_t = (
            ob[:, t * fo:(t + 1) * fo]
            + U[:, 0:fo]
            + amp * U[:, fo:2 * fo]
            + att * U[:, 2 * fo:3 * fo]
        )
        outs.append(o_t)
    o = jnp.concatenate(outs, axis=-1)
    o = jnp.dot(o, linw_ref[...], preferred_element_type=jnp.float32) + linb_ref[...]
    if with_bn:
        o = o * (1.0 / np.sqrt(1.0 + 1e-5)) * gamma_ref[...] + beta_ref[...]
        o = jnp.maximum(o, 0.0)
    o_ref[...] = o


def _post(h, A, SB, SQ, MN, MX, cnt, w0, wc, preb, postb, linw, linb, gamma, beta,
          with_bn, bn=512):
    n = h.shape[0]
    fo = w0.shape[1] // 2
    fout = linw.shape[1]
    grid = (n // bn,)
    row = lambda shp: pl.BlockSpec(shp, lambda i: (i, 0))
    full2 = lambda shp: pl.BlockSpec(shp, lambda i: (0, 0))
    return pl.pallas_call(
        functools.partial(_post_body, fo, with_bn),
        grid=grid,
        in_specs=[
            row((bn, 128)),
            row((bn, 256)),
            row((bn, 256)),
            row((bn, 256)),
            row((bn, 256)),
            row((bn, 256)),
            row((bn, 1)),
            full2((128, 2 * fo)),
            pl.BlockSpec((2, 512, 3 * fo), lambda i: (0, 0, 0)),
            full2((1, 256)),
            full2((1, 2 * fo)),
            full2((fout, fout)),
            full2((1, fout)),
            full2((1, fout)),
            full2((1, fout)),
        ],
        out_specs=row((bn, fout)),
        out_shape=jax.ShapeDtypeStruct((n, fout), jnp.float32),
    )(h, A, SB, SQ, MN, MX, cnt, w0, wc, preb, postb, linw, linb, gamma, beta)


def _conv(h, lists, cnts, pre_W, pre_b, post_W, post_b, lin_W, lin_b,
          gamma, beta, with_bn):
    F = 128
    fo = post_W.shape[2]
    Wd = jnp.concatenate([pre_W[0][:F], pre_W[1][:F]], axis=1)
    Ws = jnp.concatenate([pre_W[0][F:], pre_W[1][F:]], axis=1)
    A = _mm(h, Wd)
    B = _mm(h, Ws)
    bcat = B.reshape(NPAD, 4, 64).transpose(1, 0, 2).reshape(4 * NPAD, 64)
    SB, SQ, MN, MX, cnt16 = _scatter_stats(lists, cnts, bcat)
    cnt = cnt16[:, 0:1]
    w0 = jnp.concatenate([post_W[0][:F], post_W[1][:F]], axis=1)
    wc = jnp.stack([
        jnp.concatenate([post_W[t][F:F + 512], post_W[t][F + 512:F + 1024],
                         post_W[t][F + 1024:]], axis=1)
        for t in range(2)
    ])
    preb = jnp.concatenate([pre_b[0], pre_b[1]])[None, :]
    postb = jnp.concatenate([post_b[0], post_b[1]])[None, :]
    fout = lin_W.shape[1]
    if gamma is None:
        gamma = jnp.ones((fout,), jnp.float32)
        beta = jnp.zeros((fout,), jnp.float32)
    return _post(h, A, SB, SQ, MN, MX, cnt, w0, wc, preb, postb, lin_W,
                 lin_b.reshape(1, -1), gamma.reshape(1, -1), beta.reshape(1, -1), with_bn)


def kernel(x, edge_index, emb_W, emb_b, pre_W1, pre_b1, post_W1, post_b1, lin_W1,
           lin_b1, bn_gamma, bn_beta, pre_W2, pre_b2, post_W2, post_b2, lin_W2, lin_b2):
    n = x.shape[0]
    src = edge_index[0]
    dst = edge_index[1]
    lists, cnts = _bin_edges(src, dst)
    x_pad = jnp.pad(x, ((0, NPAD - n), (0, 0)))
    h = _mm(x_pad, emb_W, emb_b)
    h = _conv(h, lists, cnts, pre_W1, pre_b1, post_W1, post_b1, lin_W1, lin_b1,
              bn_gamma, bn_beta, True)
    h = _conv(h, lists, cnts, pre_W2, pre_b2, post_W2, post_b2, lin_W2, lin_b2,
              None, None, False)
    return h[:n]
